# baseline (speedup 1.0000x reference)
import functools

import jax
import jax.numpy as jnp
from jax import lax
from jax.experimental import pallas as pl
from jax.experimental.pallas import tpu as pltpu
from jax.experimental.pallas import tpu_sc as plsc

N = 100000
E = 6400000

NC = 2
NS = 16
NW = NC * NS
LANES = 16

NP = 100352
SLICE = NP // NS
ROWS = NP // 128

EROWS = E // 128
CR = 32
NCHUNK = 48
WB = 10
RB = 1568
RS = 1560
TAIL = 32

_mesh = functools.partial(
    plsc.VectorSubcoreMesh, core_axis_name="c", subcore_axis_name="s")


def _worker_rows(w):
    start = jnp.where(w < WB, w * RB, WB * RB + (w - WB) * RS)
    nrows = jnp.where(w < WB, RB, RS)
    return start, nrows


def _sc_deg_body(row_h, ew_h, z_h, degp_h, deg_sh,
                 rowb, ewb, sla, slb, ss):
    cid = lax.axis_index("c")
    sid = lax.axis_index("s")
    w = sid * NC + cid
    sl = pl.ds(sid * SLICE, SLICE)
    pltpu.sync_copy(z_h.at[sl], deg_sh.at[sl])
    plsc.subcore_barrier()

    start, nrows = _worker_rows(w)
    lsems = (sla, slb)

    def lin_issue(g, p):
        r0 = start + g * CR
        pltpu.async_copy(row_h.at[pl.ds(r0, CR)], rowb.at[p], lsems[p])
        pltpu.async_copy(ew_h.at[pl.ds(r0, CR)], ewb.at[p], lsems[p])

    def lin_wait(p):
        pltpu.make_async_copy(row_h.at[pl.ds(0, CR)], rowb.at[p], lsems[p]).wait()
        pltpu.make_async_copy(ew_h.at[pl.ds(0, CR)], ewb.at[p], lsems[p]).wait()

    def do_chunk(g, p):
        lin_wait(p)

        @pl.when(g + 1 < NCHUNK)
        def _():
            lin_issue(g + 1, 1 - p)

        def srow(j, c2):
            pltpu.async_copy(ewb.at[p, j], deg_sh.at[rowb.at[p, j]], ss,
                             add=True)
            return c2

        lax.fori_loop(0, CR, srow, 0)
        pltpu.make_async_copy(ew_h.at[pl.ds(0, CR)], ewb.at[p], ss).wait()

    lin_issue(0, 0)

    def pair(t, carry):
        do_chunk(2 * t, 0)
        do_chunk(2 * t + 1, 1)
        return carry

    lax.fori_loop(0, NCHUNK // 2, pair, 0)

    tstart = start + nrows - TAIL
    skip = TAIL - (nrows - NCHUNK * CR)
    pltpu.sync_copy(row_h.at[pl.ds(tstart, TAIL)], rowb.at[0])
    pltpu.sync_copy(ew_h.at[pl.ds(tstart, TAIL)], ewb.at[0])

    def trow(j, c2):
        pltpu.sync_copy(ewb.at[0, j], deg_sh.at[rowb.at[0, j]], add=True)
        return c2

    lax.fori_loop(skip, TAIL, trow, 0)
    plsc.subcore_barrier()
    pltpu.sync_copy(deg_sh.at[sl], degp_h.at[cid, sl])


def _sc_pass_body(row_h, col_h, ew_h, yq_h, z_h, acc0p_h, acc1p_h,
                  yq_sh, acc0_sh, acc1_sh,
                  rowb, colb, ewb, ygi, uub,
                  sla, slb, sga, sgb, sgc, sgd, ss):
    cid = lax.axis_index("c")
    sid = lax.axis_index("s")
    w = sid * NC + cid
    sl = pl.ds(sid * SLICE, SLICE)
    pltpu.sync_copy(yq_h.at[sl], yq_sh.at[sl])
    pltpu.sync_copy(z_h.at[sl], acc0_sh.at[sl])
    pltpu.sync_copy(z_h.at[sl], acc1_sh.at[sl])
    plsc.subcore_barrier()

    start, nrows = _worker_rows(w)
    lsems = (sla, slb)

    def lin_issue(g, p):
        r0 = start + g * CR
        pltpu.async_copy(row_h.at[pl.ds(r0, CR)], rowb.at[p], lsems[p])
        pltpu.async_copy(col_h.at[pl.ds(r0, CR)], colb.at[p], lsems[p])
        pltpu.async_copy(ew_h.at[pl.ds(r0, CR)], ewb.at[p], lsems[p])

    def lin_wait(p):
        pltpu.make_async_copy(row_h.at[pl.ds(0, CR)], rowb.at[p], lsems[p]).wait()
        pltpu.make_async_copy(col_h.at[pl.ds(0, CR)], colb.at[p], lsems[p]).wait()
        pltpu.make_async_copy(ew_h.at[pl.ds(0, CR)], ewb.at[p], lsems[p]).wait()

    def do_chunk(g, p):
        lin_wait(p)

        @pl.when(g + 1 < NCHUNK)
        def _():
            lin_issue(g + 1, 1 - p)

        cb = colb.at[p]
        eb = ewb.at[p]
        rb = rowb.at[p]

        def gat_issue(j, sem):
            pltpu.async_copy(yq_sh.at[cb.at[j]], ygi.at[j], sem)

        def gat_wait(j, sem):
            pltpu.make_async_copy(yq_sh.at[cb.at[j]], ygi.at[j], sem).wait()

        hi_mask = jnp.full((LANES,), -65536, jnp.int32)

        def work(j):
            for q in range(128 // LANES):
                s = pl.ds(q * LANES, LANES)
                ev = eb[j, s]
                yq = ygi[j, s]
                y0 = lax.bitcast_convert_type(yq & hi_mask, jnp.float32)
                y1 = lax.bitcast_convert_type(
                    lax.shift_left(yq, jnp.full((LANES,), 16, jnp.int32)),
                    jnp.float32)
                uub[2 * j, s] = -(ev * y0)
                uub[2 * j + 1, s] = -(ev * y1)
            pltpu.async_copy(uub.at[2 * j], acc0_sh.at[rb.at[j]], ss,
                             add=True)
            pltpu.async_copy(uub.at[2 * j + 1], acc1_sh.at[rb.at[j]], ss,
                             add=True)

        gsems = (sga, sgb, sgc, sgd)
        for r in range(4):
            gat_issue(r, gsems[r])

        def srow(t, c2):
            for r in range(4):
                j = 4 * t + r
                gat_wait(j, gsems[r])
                gat_issue(j + 4, gsems[r])
                work(j)
            return c2

        lax.fori_loop(0, CR // 4 - 1, srow, 0)
        for r in range(4):
            j = CR - 4 + r
            gat_wait(j, gsems[r])
            work(j)

        pltpu.make_async_copy(ew_h.at[pl.ds(0, 2 * CR)], uub, ss).wait()

    lin_issue(0, 0)

    def pair(t, carry):
        do_chunk(2 * t, 0)
        do_chunk(2 * t + 1, 1)
        return carry

    lax.fori_loop(0, NCHUNK // 2, pair, 0)

    tstart = start + nrows - TAIL
    skip = TAIL - (nrows - NCHUNK * CR)
    pltpu.sync_copy(row_h.at[pl.ds(tstart, TAIL)], rowb.at[0])
    pltpu.sync_copy(col_h.at[pl.ds(tstart, TAIL)], colb.at[0])
    pltpu.sync_copy(ew_h.at[pl.ds(tstart, TAIL)], ewb.at[0])
    hi_mask_t = jnp.full((LANES,), -65536, jnp.int32)
    sh16 = jnp.full((LANES,), 16, jnp.int32)

    def trow(j, c2):
        pltpu.sync_copy(yq_sh.at[colb.at[0, j]], ygi.at[j])
        for q in range(128 // LANES):
            s = pl.ds(q * LANES, LANES)
            ev = ewb[0, j, s]
            yq = ygi[j, s]
            y0 = lax.bitcast_convert_type(yq & hi_mask_t, jnp.float32)
            y1 = lax.bitcast_convert_type(lax.shift_left(yq, sh16),
                                          jnp.float32)
            uub[2 * j, s] = -(ev * y0)
            uub[2 * j + 1, s] = -(ev * y1)
        pltpu.sync_copy(uub.at[2 * j], acc0_sh.at[rowb.at[0, j]], add=True)
        pltpu.sync_copy(uub.at[2 * j + 1], acc1_sh.at[rowb.at[0, j]],
                        add=True)
        return c2

    lax.fori_loop(skip, TAIL, trow, 0)
    plsc.subcore_barrier()
    pltpu.sync_copy(acc0_sh.at[sl], acc0p_h.at[cid, sl])
    pltpu.sync_copy(acc1_sh.at[sl], acc1p_h.at[cid, sl])


_sc_deg = pl.kernel(
    _sc_deg_body,
    out_type=jax.ShapeDtypeStruct((NC, NP), jnp.float32),
    mesh=_mesh(),
    scratch_types=[
        pltpu.VMEM_SHARED((NP,), jnp.float32),
        pltpu.VMEM((2, CR, 128), jnp.int32),
        pltpu.VMEM((2, CR, 128), jnp.float32),
        pltpu.SemaphoreType.DMA,
        pltpu.SemaphoreType.DMA,
        pltpu.SemaphoreType.DMA,
    ],
)

_sc_pass = pl.kernel(
    _sc_pass_body,
    out_type=(jax.ShapeDtypeStruct((NC, NP), jnp.float32),
              jax.ShapeDtypeStruct((NC, NP), jnp.float32)),
    mesh=_mesh(),
    scratch_types=[
        pltpu.VMEM_SHARED((NP,), jnp.int32),
        pltpu.VMEM_SHARED((NP,), jnp.float32),
        pltpu.VMEM_SHARED((NP,), jnp.float32),
        pltpu.VMEM((2, CR, 128), jnp.int32),
        pltpu.VMEM((2, CR, 128), jnp.int32),
        pltpu.VMEM((2, CR, 128), jnp.float32),
        pltpu.VMEM((CR, 128), jnp.int32),
        pltpu.VMEM((2 * CR, 128), jnp.float32),
        pltpu.SemaphoreType.DMA,
        pltpu.SemaphoreType.DMA,
        pltpu.SemaphoreType.DMA,
        pltpu.SemaphoreType.DMA,
        pltpu.SemaphoreType.DMA,
        pltpu.SemaphoreType.DMA,
        pltpu.SemaphoreType.DMA,
    ],
)


def _pack2bf16(a, b):
    au = lax.bitcast_convert_type(
        a.astype(jnp.bfloat16).astype(jnp.float32), jnp.uint32)
    bu = lax.bitcast_convert_type(
        b.astype(jnp.bfloat16).astype(jnp.float32), jnp.uint32)
    return lax.bitcast_convert_type(au | (bu >> 16), jnp.int32)


def _tc1_body(deg0, deg1, x0, x1, m0, m1, n0, n1,
              dis_o, xx0_o, xx1_o, yq_o):
    deg = deg0[...] + deg1[...]
    dis = jnp.where(deg > 0, lax.rsqrt(jnp.maximum(deg, 1e-12)), 0.0)
    xx0 = m0[...] * x0[...] + (1.0 - m0[...]) * n0[...]
    xx1 = m1[...] * x1[...] + (1.0 - m1[...]) * n1[...]
    dis_o[...] = dis
    xx0_o[...] = xx0
    xx1_o[...] = xx1
    yq_o[...] = _pack2bf16(dis * xx0, dis * xx1)


def _tc2_body(a00, a01, a10, a11, dis, t10_o, t11_o, zq_o):
    d = dis[...]
    t10 = d * (a00[...] + a01[...])
    t11 = d * (a10[...] + a11[...])
    t10_o[...] = t10
    t11_o[...] = t11
    zq_o[...] = _pack2bf16(d * t10, d * t11)


def _tc3_body(b00, b01, b10, b11, dis, xx0, xx1, t10, t11, m0, m1, wv,
              r0_o, r1_o, i0_o, i1_o):
    d = dis[...]
    X0, X1 = xx0[...], xx1[...]
    T10, T11 = t10[...], t11[...]
    T20 = 2.0 * d * (b00[...] + b01[...]) - X0
    T21 = 2.0 * d * (b10[...] + b11[...]) - X1
    outz = (X0 * wv[0] + X1 * wv[1] + T10 * wv[2] + T11 * wv[3]
            + T20 * wv[4] + T21 * wv[5] + wv[6])
    outh = (X0 * wv[7] + X1 * wv[8] + T10 * wv[9] + T11 * wv[10]
            + T20 * wv[11] + T21 * wv[12] + wv[13])
    Z = jax.nn.sigmoid(outz)
    Ht = jnp.tanh(outh)
    H = (1.0 - Z) * Ht
    i0 = jax.nn.sigmoid(H * wv[14] + wv[16])
    i1 = jax.nn.sigmoid(H * wv[15] + wv[17])
    M0, M1 = m0[...], m1[...]
    i0_o[...] = i0
    i1_o[...] = i1
    r0_o[...] = M0 * X0 + (1.0 - M0) * i0
    r1_o[...] = M1 * X1 + (1.0 - M1) * i1


_f32 = jnp.float32
_blk = jax.ShapeDtypeStruct((ROWS, 128), _f32)
_blki = jax.ShapeDtypeStruct((ROWS, 128), jnp.int32)

_tc1 = pl.pallas_call(_tc1_body, out_shape=(_blk,) * 3 + (_blki,))
_tc2 = pl.pallas_call(_tc2_body, out_shape=(_blk,) * 2 + (_blki,))
_tc3 = pl.pallas_call(
    _tc3_body,
    out_shape=(_blk,) * 4,
    in_specs=[pl.BlockSpec(memory_space=pltpu.VMEM)] * 11
    + [pl.BlockSpec(memory_space=pltpu.SMEM)],
)


def _padn(a):
    return jnp.concatenate([a, jnp.zeros((NP - N,), a.dtype)]).reshape(ROWS, 128)


def kernel(x, input_mask, edge_index, edge_weights, noise,
           Wxz, bxz, Whz, bhz, Wxr, bxr, Whr, bhr,
           Wxh, bxh, Whh, bhh, Wfc, bfc):
    rowp = edge_index[0].reshape(EROWS, 128)
    colp = edge_index[1].reshape(EROWS, 128)
    ewp = edge_weights.reshape(EROWS, 128)

    x0, x1 = _padn(x[:, 0]), _padn(x[:, 1])
    m0, m1 = _padn(input_mask[:, 0]), _padn(input_mask[:, 1])
    n0, n1 = _padn(noise[:, 0]), _padn(noise[:, 1])

    z1 = jnp.zeros((NP,), _f32)

    degp = _sc_deg(rowp, ewp, z1)
    dis, xx0, xx1, yq = _tc1(
        degp[0].reshape(ROWS, 128), degp[1].reshape(ROWS, 128),
        x0, x1, m0, m1, n0, n1)

    acc0p, acc1p = _sc_pass(rowp, colp, ewp, yq.reshape(NP), z1)
    t10, t11, zq = _tc2(
        acc0p[0].reshape(ROWS, 128), acc0p[1].reshape(ROWS, 128),
        acc1p[0].reshape(ROWS, 128), acc1p[1].reshape(ROWS, 128),
        dis)

    b0p, b1p = _sc_pass(rowp, colp, ewp, zq.reshape(NP), z1)

    wv = jnp.stack([
        Wxz[0, 0, 0], Wxz[0, 1, 0], Wxz[1, 0, 0], Wxz[1, 1, 0],
        Wxz[2, 0, 0], Wxz[2, 1, 0], bxz[0] + bhz[0],
        Wxh[0, 0, 0], Wxh[0, 1, 0], Wxh[1, 0, 0], Wxh[1, 1, 0],
        Wxh[2, 0, 0], Wxh[2, 1, 0], bxh[0] + bhh[0],
        Wfc[0, 0], Wfc[0, 1], bfc[0], bfc[1],
    ])
    r0, r1, i0, i1 = _tc3(
        b0p[0].reshape(ROWS, 128), b0p[1].reshape(ROWS, 128),
        b1p[0].reshape(ROWS, 128), b1p[1].reshape(ROWS, 128),
        dis, xx0, xx1, t10, t11, m0, m1, wv)

    res = jnp.stack([r0.reshape(NP)[:N], r1.reshape(NP)[:N]], axis=1)
    imp = jnp.stack([i0.reshape(NP)[:N], i1.reshape(NP)[:N]], axis=1)
    return (res, imp)


# speedup vs baseline: 1.1286x; 1.1286x over previous
import functools

import jax
import jax.numpy as jnp
from jax import lax
from jax.experimental import pallas as pl
from jax.experimental.pallas import tpu as pltpu
from jax.experimental.pallas import tpu_sc as plsc

N = 100000
E = 6400000

NC = 2
NS = 16
NW = NC * NS
LANES = 16

NP = 100352
SLICE = NP // NS
ROWS = NP // 128

EROWS = E // 128
CR = 32
NCHUNK = 48
WB = 10
RB = 1568
RS = 1560
TAIL = 32

_mesh = functools.partial(
    plsc.VectorSubcoreMesh, core_axis_name="c", subcore_axis_name="s")


def _worker_rows(w):
    start = jnp.where(w < WB, w * RB, WB * RB + (w - WB) * RS)
    nrows = jnp.where(w < WB, RB, RS)
    return start, nrows


def _sc_deg_body(row_h, ew_h, z_h, degp_h, deg_sh,
                 rowb, ewb, sla, slb, ss):
    cid = lax.axis_index("c")
    sid = lax.axis_index("s")
    w = sid * NC + cid
    sl = pl.ds(sid * SLICE, SLICE)
    pltpu.sync_copy(z_h.at[sl], deg_sh.at[sl])
    plsc.subcore_barrier()

    start, nrows = _worker_rows(w)
    lsems = (sla, slb)

    def lin_issue(g, p):
        r0 = start + g * CR
        pltpu.async_copy(row_h.at[pl.ds(r0, CR)], rowb.at[p], lsems[p])
        pltpu.async_copy(ew_h.at[pl.ds(r0, CR)], ewb.at[p], lsems[p])

    def lin_wait(p):
        pltpu.make_async_copy(row_h.at[pl.ds(0, CR)], rowb.at[p], lsems[p]).wait()
        pltpu.make_async_copy(ew_h.at[pl.ds(0, CR)], ewb.at[p], lsems[p]).wait()

    def do_chunk(g, p):
        lin_wait(p)

        @pl.when(g + 1 < NCHUNK)
        def _():
            lin_issue(g + 1, 1 - p)

        def srow(j, c2):
            pltpu.async_copy(ewb.at[p, j], deg_sh.at[rowb.at[p, j]], ss,
                             add=True)
            return c2

        lax.fori_loop(0, CR, srow, 0)
        pltpu.make_async_copy(ew_h.at[pl.ds(0, CR)], ewb.at[p], ss).wait()

    lin_issue(0, 0)

    def pair(t, carry):
        do_chunk(2 * t, 0)
        do_chunk(2 * t + 1, 1)
        return carry

    lax.fori_loop(0, NCHUNK // 2, pair, 0)

    tstart = start + nrows - TAIL
    skip = TAIL - (nrows - NCHUNK * CR)
    pltpu.sync_copy(row_h.at[pl.ds(tstart, TAIL)], rowb.at[0])
    pltpu.sync_copy(ew_h.at[pl.ds(tstart, TAIL)], ewb.at[0])

    def trow(j, c2):
        pltpu.sync_copy(ewb.at[0, j], deg_sh.at[rowb.at[0, j]], add=True)
        return c2

    lax.fori_loop(skip, TAIL, trow, 0)
    plsc.subcore_barrier()
    pltpu.sync_copy(deg_sh.at[sl], degp_h.at[cid, sl])


def _sc_pass_body(row_h, col_h, ew_h, yq_h, z_h, acc0p_h, acc1p_h,
                  yq_sh, acc0_sh, acc1_sh,
                  rowb, colb, ewb, ygi, uub,
                  sla, slb, sga, sgb, ss):
    cid = lax.axis_index("c")
    sid = lax.axis_index("s")
    w = sid * NC + cid
    sl = pl.ds(sid * SLICE, SLICE)
    pltpu.sync_copy(yq_h.at[sl], yq_sh.at[sl])
    pltpu.sync_copy(z_h.at[sl], acc0_sh.at[sl])
    pltpu.sync_copy(z_h.at[sl], acc1_sh.at[sl])
    plsc.subcore_barrier()

    start, nrows = _worker_rows(w)
    lsems = (sla, slb)

    def lin_issue(g, p):
        r0 = start + g * CR
        pltpu.async_copy(row_h.at[pl.ds(r0, CR)], rowb.at[p], lsems[p])
        pltpu.async_copy(col_h.at[pl.ds(r0, CR)], colb.at[p], lsems[p])
        pltpu.async_copy(ew_h.at[pl.ds(r0, CR)], ewb.at[p], lsems[p])

    def lin_wait(p):
        pltpu.make_async_copy(row_h.at[pl.ds(0, CR)], rowb.at[p], lsems[p]).wait()
        pltpu.make_async_copy(col_h.at[pl.ds(0, CR)], colb.at[p], lsems[p]).wait()
        pltpu.make_async_copy(ew_h.at[pl.ds(0, CR)], ewb.at[p], lsems[p]).wait()

    def do_chunk(g, p):
        lin_wait(p)

        @pl.when(g + 1 < NCHUNK)
        def _():
            lin_issue(g + 1, 1 - p)

        cb = colb.at[p]
        eb = ewb.at[p]
        rb = rowb.at[p]

        def gat_issue(j, sem):
            pltpu.async_copy(yq_sh.at[cb.at[j]], ygi.at[j], sem)

        def gat_wait(j, sem):
            pltpu.make_async_copy(yq_sh.at[cb.at[j]], ygi.at[j], sem).wait()

        hi_mask = jnp.full((LANES,), -65536, jnp.int32)

        def work(j):
            for q in range(128 // LANES):
                s = pl.ds(q * LANES, LANES)
                ev = eb[j, s]
                yq = ygi[j, s]
                y0 = lax.bitcast_convert_type(yq & hi_mask, jnp.float32)
                y1 = lax.bitcast_convert_type(
                    lax.shift_left(yq, jnp.full((LANES,), 16, jnp.int32)),
                    jnp.float32)
                uub[2 * j, s] = -(ev * y0)
                uub[2 * j + 1, s] = -(ev * y1)
            pltpu.async_copy(uub.at[2 * j], acc0_sh.at[rb.at[j]], ss,
                             add=True)
            pltpu.async_copy(uub.at[2 * j + 1], acc1_sh.at[rb.at[j]], ss,
                             add=True)

        gat_issue(0, sga)
        gat_issue(1, sgb)

        def srow(t, c2):
            ja = 2 * t
            gat_wait(ja, sga)
            gat_issue(ja + 2, sga)
            work(ja)
            gat_wait(ja + 1, sgb)
            gat_issue(ja + 3, sgb)
            work(ja + 1)
            return c2

        lax.fori_loop(0, CR // 2 - 1, srow, 0)
        gat_wait(CR - 2, sga)
        work(CR - 2)
        gat_wait(CR - 1, sgb)
        work(CR - 1)

        pltpu.make_async_copy(ew_h.at[pl.ds(0, 2 * CR)], uub, ss).wait()

    lin_issue(0, 0)

    def pair(t, carry):
        do_chunk(2 * t, 0)
        do_chunk(2 * t + 1, 1)
        return carry

    lax.fori_loop(0, NCHUNK // 2, pair, 0)

    tstart = start + nrows - TAIL
    skip = TAIL - (nrows - NCHUNK * CR)
    pltpu.sync_copy(row_h.at[pl.ds(tstart, TAIL)], rowb.at[0])
    pltpu.sync_copy(col_h.at[pl.ds(tstart, TAIL)], colb.at[0])
    pltpu.sync_copy(ew_h.at[pl.ds(tstart, TAIL)], ewb.at[0])
    hi_mask_t = jnp.full((LANES,), -65536, jnp.int32)
    sh16 = jnp.full((LANES,), 16, jnp.int32)

    def trow(j, c2):
        pltpu.sync_copy(yq_sh.at[colb.at[0, j]], ygi.at[j])
        for q in range(128 // LANES):
            s = pl.ds(q * LANES, LANES)
            ev = ewb[0, j, s]
            yq = ygi[j, s]
            y0 = lax.bitcast_convert_type(yq & hi_mask_t, jnp.float32)
            y1 = lax.bitcast_convert_type(lax.shift_left(yq, sh16),
                                          jnp.float32)
            uub[2 * j, s] = -(ev * y0)
            uub[2 * j + 1, s] = -(ev * y1)
        pltpu.sync_copy(uub.at[2 * j], acc0_sh.at[rowb.at[0, j]], add=True)
        pltpu.sync_copy(uub.at[2 * j + 1], acc1_sh.at[rowb.at[0, j]],
                        add=True)
        return c2

    lax.fori_loop(skip, TAIL, trow, 0)
    plsc.subcore_barrier()
    pltpu.sync_copy(acc0_sh.at[sl], acc0p_h.at[cid, sl])
    pltpu.sync_copy(acc1_sh.at[sl], acc1p_h.at[cid, sl])


_sc_deg = pl.kernel(
    _sc_deg_body,
    out_type=jax.ShapeDtypeStruct((NC, NP), jnp.float32),
    mesh=_mesh(),
    scratch_types=[
        pltpu.VMEM_SHARED((NP,), jnp.float32),
        pltpu.VMEM((2, CR, 128), jnp.int32),
        pltpu.VMEM((2, CR, 128), jnp.float32),
        pltpu.SemaphoreType.DMA,
        pltpu.SemaphoreType.DMA,
        pltpu.SemaphoreType.DMA,
    ],
)

_sc_pass = pl.kernel(
    _sc_pass_body,
    out_type=(jax.ShapeDtypeStruct((NC, NP), jnp.float32),
              jax.ShapeDtypeStruct((NC, NP), jnp.float32)),
    mesh=_mesh(),
    scratch_types=[
        pltpu.VMEM_SHARED((NP,), jnp.int32),
        pltpu.VMEM_SHARED((NP,), jnp.float32),
        pltpu.VMEM_SHARED((NP,), jnp.float32),
        pltpu.VMEM((2, CR, 128), jnp.int32),
        pltpu.VMEM((2, CR, 128), jnp.int32),
        pltpu.VMEM((2, CR, 128), jnp.float32),
        pltpu.VMEM((CR, 128), jnp.int32),
        pltpu.VMEM((2 * CR, 128), jnp.float32),
        pltpu.SemaphoreType.DMA,
        pltpu.SemaphoreType.DMA,
        pltpu.SemaphoreType.DMA,
        pltpu.SemaphoreType.DMA,
        pltpu.SemaphoreType.DMA,
    ],
)


def _pack2bf16(a, b):
    au = lax.bitcast_convert_type(
        a.astype(jnp.bfloat16).astype(jnp.float32), jnp.uint32)
    bu = lax.bitcast_convert_type(
        b.astype(jnp.bfloat16).astype(jnp.float32), jnp.uint32)
    return lax.bitcast_convert_type(au | (bu >> 16), jnp.int32)


def _tc1_body(deg0, deg1, x0, x1, m0, m1, n0, n1,
              dis_o, xx0_o, xx1_o, yq_o):
    deg = deg0[...] + deg1[...]
    dis = jnp.where(deg > 0, lax.rsqrt(jnp.maximum(deg, 1e-12)), 0.0)
    xx0 = m0[...] * x0[...] + (1.0 - m0[...]) * n0[...]
    xx1 = m1[...] * x1[...] + (1.0 - m1[...]) * n1[...]
    dis_o[...] = dis
    xx0_o[...] = xx0
    xx1_o[...] = xx1
    yq_o[...] = _pack2bf16(dis * xx0, dis * xx1)


def _tc2_body(a00, a01, a10, a11, dis, t10_o, t11_o, zq_o):
    d = dis[...]
    t10 = d * (a00[...] + a01[...])
    t11 = d * (a10[...] + a11[...])
    t10_o[...] = t10
    t11_o[...] = t11
    zq_o[...] = _pack2bf16(d * t10, d * t11)


def _tc3_body(b00, b01, b10, b11, dis, xx0, xx1, t10, t11, m0, m1, wv,
              r0_o, r1_o, i0_o, i1_o):
    d = dis[...]
    X0, X1 = xx0[...], xx1[...]
    T10, T11 = t10[...], t11[...]
    T20 = 2.0 * d * (b00[...] + b01[...]) - X0
    T21 = 2.0 * d * (b10[...] + b11[...]) - X1
    outz = (X0 * wv[0] + X1 * wv[1] + T10 * wv[2] + T11 * wv[3]
            + T20 * wv[4] + T21 * wv[5] + wv[6])
    outh = (X0 * wv[7] + X1 * wv[8] + T10 * wv[9] + T11 * wv[10]
            + T20 * wv[11] + T21 * wv[12] + wv[13])
    Z = jax.nn.sigmoid(outz)
    Ht = jnp.tanh(outh)
    H = (1.0 - Z) * Ht
    i0 = jax.nn.sigmoid(H * wv[14] + wv[16])
    i1 = jax.nn.sigmoid(H * wv[15] + wv[17])
    M0, M1 = m0[...], m1[...]
    i0_o[...] = i0
    i1_o[...] = i1
    r0_o[...] = M0 * X0 + (1.0 - M0) * i0
    r1_o[...] = M1 * X1 + (1.0 - M1) * i1


_f32 = jnp.float32
_blk = jax.ShapeDtypeStruct((ROWS, 128), _f32)
_blki = jax.ShapeDtypeStruct((ROWS, 128), jnp.int32)

_tc1 = pl.pallas_call(_tc1_body, out_shape=(_blk,) * 3 + (_blki,))
_tc2 = pl.pallas_call(_tc2_body, out_shape=(_blk,) * 2 + (_blki,))
_tc3 = pl.pallas_call(
    _tc3_body,
    out_shape=(_blk,) * 4,
    in_specs=[pl.BlockSpec(memory_space=pltpu.VMEM)] * 11
    + [pl.BlockSpec(memory_space=pltpu.SMEM)],
)


def _padn(a):
    return jnp.concatenate([a, jnp.zeros((NP - N,), a.dtype)]).reshape(ROWS, 128)


def kernel(x, input_mask, edge_index, edge_weights, noise,
           Wxz, bxz, Whz, bhz, Wxr, bxr, Whr, bhr,
           Wxh, bxh, Whh, bhh, Wfc, bfc):
    rowp = edge_index[0].reshape(EROWS, 128)
    colp = edge_index[1].reshape(EROWS, 128)
    ewp = edge_weights.reshape(EROWS, 128)

    x0, x1 = _padn(x[:, 0]), _padn(x[:, 1])
    m0, m1 = _padn(input_mask[:, 0]), _padn(input_mask[:, 1])
    n0, n1 = _padn(noise[:, 0]), _padn(noise[:, 1])

    z1 = jnp.zeros((NP,), _f32)

    degp = _sc_deg(rowp, ewp, z1)
    dis, xx0, xx1, yq = _tc1(
        degp[0].reshape(ROWS, 128), degp[1].reshape(ROWS, 128),
        x0, x1, m0, m1, n0, n1)

    acc0p, acc1p = _sc_pass(rowp, colp, ewp, yq.reshape(NP), z1)
    t10, t11, zq = _tc2(
        acc0p[0].reshape(ROWS, 128), acc0p[1].reshape(ROWS, 128),
        acc1p[0].reshape(ROWS, 128), acc1p[1].reshape(ROWS, 128),
        dis)

    b0p, b1p = _sc_pass(rowp, colp, ewp, zq.reshape(NP), z1)

    wv = jnp.stack([
        Wxz[0, 0, 0], Wxz[0, 1, 0], Wxz[1, 0, 0], Wxz[1, 1, 0],
        Wxz[2, 0, 0], Wxz[2, 1, 0], bxz[0] + bhz[0],
        Wxh[0, 0, 0], Wxh[0, 1, 0], Wxh[1, 0, 0], Wxh[1, 1, 0],
        Wxh[2, 0, 0], Wxh[2, 1, 0], bxh[0] + bhh[0],
        Wfc[0, 0], Wfc[0, 1], bfc[0], bfc[1],
    ])
    r0, r1, i0, i1 = _tc3(
        b0p[0].reshape(ROWS, 128), b0p[1].reshape(ROWS, 128),
        b1p[0].reshape(ROWS, 128), b1p[1].reshape(ROWS, 128),
        dis, xx0, xx1, t10, t11, m0, m1, wv)

    res = jnp.stack([r0.reshape(NP)[:N], r1.reshape(NP)[:N]], axis=1)
    imp = jnp.stack([i0.reshape(NP)[:N], i1.reshape(NP)[:N]], axis=1)
    return (res, imp)
